# speedup vs baseline: 4.1734x; 4.1734x over previous
import functools

import jax
import jax.numpy as jnp
from jax.experimental import pallas as pl


def _body(x_ref, wc_ref, cb_ref, eb_ref, probs_ref, final_ref, idx_ref,
          *, tb, e, k, d_out):
    acts = jnp.dot(x_ref[...], wc_ref[...], preferred_element_type=jnp.float32)
    acts = acts + cb_ref[...]
    gate_out = acts[:, 0:e]
    probs = jax.nn.sigmoid(gate_out)
    probs_ref[...] = probs
    logits = gate_out + eb_ref[...][:, 0:e]

    col = jax.lax.broadcasted_iota(jnp.int32, (tb, e), 1)
    m1 = jnp.max(logits, axis=1, keepdims=True)
    i1 = jnp.min(jnp.where(logits == m1, col, e), axis=1, keepdims=True)
    masked = jnp.where(col == i1, -jnp.inf, logits)
    m2 = jnp.max(masked, axis=1, keepdims=True)
    i2 = jnp.min(jnp.where(masked == m2, col, e), axis=1, keepdims=True)

    p1 = jnp.sum(jnp.where(col == i1, probs, 0.0), axis=1, keepdims=True)
    p2 = jnp.sum(jnp.where(col == i2, probs, 0.0), axis=1, keepdims=True)

    vcols = acts[:, e:e + e * k]
    colv = jax.lax.broadcasted_iota(jnp.int32, (tb, e * k), 1)
    v1 = jnp.sum(jnp.where(colv == k * i1, vcols, 0.0), axis=1, keepdims=True)
    v2 = jnp.sum(jnp.where(colv == k * i2 + 1, vcols, 0.0), axis=1, keepdims=True)

    c = (p1 * v1 + p2 * v2) / (p1 + p2)
    final_ref[...] = jnp.broadcast_to(c, (tb, d_out))
    idx_ref[...] = jnp.concatenate([i1, i2], axis=1)


def kernel(x, gate_W, gate_b, expert_W, expert_b, expert_biases):
    B, S, D_IN = x.shape
    E, D_OUT, _ = expert_W.shape
    K = 2
    T = B * S
    TB = 512
    NCOL = 128

    xf = x.reshape(T, D_IN)
    wproj = expert_W[:, :K, :].transpose(2, 0, 1).reshape(D_IN, E * K)
    Wc = jnp.concatenate(
        [gate_W.T, wproj, jnp.zeros((D_IN, NCOL - E - E * K), jnp.float32)],
        axis=1)
    cbias = jnp.concatenate(
        [gate_b, expert_b[:, :K].reshape(E * K),
         jnp.zeros((NCOL - E - E * K,), jnp.float32)])[None, :]
    ebias = jnp.concatenate(
        [expert_biases, jnp.zeros((NCOL - E,), jnp.float32)])[None, :]

    grid = (T // TB,)
    probs, final, idx = pl.pallas_call(
        functools.partial(_body, tb=TB, e=E, k=K, d_out=D_OUT),
        grid=grid,
        in_specs=[
            pl.BlockSpec((TB, D_IN), lambda i: (i, 0)),
            pl.BlockSpec((D_IN, NCOL), lambda i: (0, 0)),
            pl.BlockSpec((1, NCOL), lambda i: (0, 0)),
            pl.BlockSpec((1, NCOL), lambda i: (0, 0)),
        ],
        out_specs=[
            pl.BlockSpec((TB, E), lambda i: (i, 0)),
            pl.BlockSpec((TB, D_OUT), lambda i: (i, 0)),
            pl.BlockSpec((TB, K), lambda i: (i, 0)),
        ],
        out_shape=[
            jax.ShapeDtypeStruct((T, E), jnp.float32),
            jax.ShapeDtypeStruct((T, D_OUT), jnp.float32),
            jax.ShapeDtypeStruct((T, K), jnp.int32),
        ],
    )(xf, Wc, cbias, ebias)

    return (final.reshape(B, S, D_OUT),
            probs.reshape(B, S, E),
            idx.reshape(B, S, K))


# speedup vs baseline: 4.6085x; 1.1042x over previous
import functools

import jax
import jax.numpy as jnp
from jax.experimental import pallas as pl


def _body(x_ref, wc_ref, cb_ref, eb_ref, probs_ref, final_ref, idx_ref,
          *, tb, e, k, d_out):
    acts = jnp.dot(x_ref[...], wc_ref[...], preferred_element_type=jnp.float32)
    acts = acts + cb_ref[...]
    gate_out = acts[:, 0:e]
    probs = jax.nn.sigmoid(gate_out)
    probs_ref[...] = probs
    logits = gate_out + eb_ref[...][:, 0:e]

    col = jax.lax.broadcasted_iota(jnp.int32, (tb, e), 1)
    m1 = jnp.max(logits, axis=1, keepdims=True)
    i1 = jnp.min(jnp.where(logits == m1, col, e), axis=1, keepdims=True)
    masked = jnp.where(col == i1, -jnp.inf, logits)
    m2 = jnp.max(masked, axis=1, keepdims=True)
    i2 = jnp.min(jnp.where(masked == m2, col, e), axis=1, keepdims=True)

    p1 = jnp.sum(jnp.where(col == i1, probs, 0.0), axis=1, keepdims=True)
    p2 = jnp.sum(jnp.where(col == i2, probs, 0.0), axis=1, keepdims=True)

    vcols = acts[:, e:e + e * k]
    colv = jax.lax.broadcasted_iota(jnp.int32, (tb, e * k), 1)
    v1 = jnp.sum(jnp.where(colv == k * i1, vcols, 0.0), axis=1, keepdims=True)
    v2 = jnp.sum(jnp.where(colv == k * i2 + 1, vcols, 0.0), axis=1, keepdims=True)

    c = (p1 * v1 + p2 * v2) / (p1 + p2)
    final_ref[...] = jnp.broadcast_to(c, (tb, d_out))
    idx_ref[...] = jnp.concatenate([i1, i2], axis=1)


def kernel(x, gate_W, gate_b, expert_W, expert_b, expert_biases):
    B, S, D_IN = x.shape
    E, D_OUT, _ = expert_W.shape
    K = 2
    T = B * S
    TB = 1024
    NCOL = 128

    xf = x.reshape(T, D_IN)
    wproj = expert_W[:, :K, :].transpose(2, 0, 1).reshape(D_IN, E * K)
    Wc = jnp.concatenate(
        [gate_W.T, wproj, jnp.zeros((D_IN, NCOL - E - E * K), jnp.float32)],
        axis=1)
    cbias = jnp.concatenate(
        [gate_b, expert_b[:, :K].reshape(E * K),
         jnp.zeros((NCOL - E - E * K,), jnp.float32)])[None, :]
    ebias = jnp.concatenate(
        [expert_biases, jnp.zeros((NCOL - E,), jnp.float32)])[None, :]

    grid = (T // TB,)
    probs, final, idx = pl.pallas_call(
        functools.partial(_body, tb=TB, e=E, k=K, d_out=D_OUT),
        grid=grid,
        in_specs=[
            pl.BlockSpec((TB, D_IN), lambda i: (i, 0)),
            pl.BlockSpec((D_IN, NCOL), lambda i: (0, 0)),
            pl.BlockSpec((1, NCOL), lambda i: (0, 0)),
            pl.BlockSpec((1, NCOL), lambda i: (0, 0)),
        ],
        out_specs=[
            pl.BlockSpec((TB, E), lambda i: (i, 0)),
            pl.BlockSpec((TB, D_OUT), lambda i: (i, 0)),
            pl.BlockSpec((TB, K), lambda i: (i, 0)),
        ],
        out_shape=[
            jax.ShapeDtypeStruct((T, E), jnp.float32),
            jax.ShapeDtypeStruct((T, D_OUT), jnp.float32),
            jax.ShapeDtypeStruct((T, K), jnp.int32),
        ],
    )(xf, Wc, cbias, ebias)

    return (final.reshape(B, S, D_OUT),
            probs.reshape(B, S, E),
            idx.reshape(B, S, K))


# speedup vs baseline: 4.8766x; 1.0582x over previous
import functools

import jax
import jax.numpy as jnp
from jax.experimental import pallas as pl


def _body(x_ref, wc_ref, cb_ref, eb_ref, probs_ref, final_ref, idx_ref,
          *, tb, e, k, d_out):
    acts = jnp.dot(x_ref[...], wc_ref[...], preferred_element_type=jnp.float32)
    acts = acts + cb_ref[...]
    gate_out = acts[:, 0:e]
    probs = jax.nn.sigmoid(gate_out)
    probs_ref[...] = probs
    logits = gate_out + eb_ref[...][:, 0:e]

    col = jax.lax.broadcasted_iota(jnp.int32, (tb, e), 1)
    m1 = jnp.max(logits, axis=1, keepdims=True)
    i1 = jnp.min(jnp.where(logits == m1, col, e), axis=1, keepdims=True)
    masked = jnp.where(col == i1, -jnp.inf, logits)
    m2 = jnp.max(masked, axis=1, keepdims=True)
    i2 = jnp.min(jnp.where(masked == m2, col, e), axis=1, keepdims=True)

    p1 = jnp.sum(jnp.where(col == i1, probs, 0.0), axis=1, keepdims=True)
    p2 = jnp.sum(jnp.where(col == i2, probs, 0.0), axis=1, keepdims=True)

    vcols = acts[:, e:e + e * k]
    colv = jax.lax.broadcasted_iota(jnp.int32, (tb, e * k), 1)
    v1 = jnp.sum(jnp.where(colv == k * i1, vcols, 0.0), axis=1, keepdims=True)
    v2 = jnp.sum(jnp.where(colv == k * i2 + 1, vcols, 0.0), axis=1, keepdims=True)

    c = (p1 * v1 + p2 * v2) / (p1 + p2)
    final_ref[...] = jnp.broadcast_to(c, (tb, d_out))
    idx_ref[...] = jnp.concatenate([i1, i2], axis=1)


def kernel(x, gate_W, gate_b, expert_W, expert_b, expert_biases):
    B, S, D_IN = x.shape
    E, D_OUT, _ = expert_W.shape
    K = 2
    T = B * S
    TB = 2048
    NCOL = 128

    xf = x.reshape(T, D_IN)
    wproj = expert_W[:, :K, :].transpose(2, 0, 1).reshape(D_IN, E * K)
    Wc = jnp.concatenate(
        [gate_W.T, wproj, jnp.zeros((D_IN, NCOL - E - E * K), jnp.float32)],
        axis=1)
    cbias = jnp.concatenate(
        [gate_b, expert_b[:, :K].reshape(E * K),
         jnp.zeros((NCOL - E - E * K,), jnp.float32)])[None, :]
    ebias = jnp.concatenate(
        [expert_biases, jnp.zeros((NCOL - E,), jnp.float32)])[None, :]

    grid = (T // TB,)
    probs, final, idx = pl.pallas_call(
        functools.partial(_body, tb=TB, e=E, k=K, d_out=D_OUT),
        grid=grid,
        in_specs=[
            pl.BlockSpec((TB, D_IN), lambda i: (i, 0)),
            pl.BlockSpec((D_IN, NCOL), lambda i: (0, 0)),
            pl.BlockSpec((1, NCOL), lambda i: (0, 0)),
            pl.BlockSpec((1, NCOL), lambda i: (0, 0)),
        ],
        out_specs=[
            pl.BlockSpec((TB, E), lambda i: (i, 0)),
            pl.BlockSpec((TB, D_OUT), lambda i: (i, 0)),
            pl.BlockSpec((TB, K), lambda i: (i, 0)),
        ],
        out_shape=[
            jax.ShapeDtypeStruct((T, E), jnp.float32),
            jax.ShapeDtypeStruct((T, D_OUT), jnp.float32),
            jax.ShapeDtypeStruct((T, K), jnp.int32),
        ],
    )(xf, Wc, cbias, ebias)

    return (final.reshape(B, S, D_OUT),
            probs.reshape(B, S, E),
            idx.reshape(B, S, K))
